# baseline (speedup 1.0000x reference)
import functools

import jax
import jax.numpy as jnp
from jax import lax
from jax.experimental import pallas as pl
from jax.experimental.pallas import tpu as pltpu
from jax.experimental.pallas import tpu_sc as plsc

NT = 16
LANES = 16
CH = 128


def _round_up(a, b):
  return (a + b - 1) // b * b


def _deg_call(npad, rpt, cpt, ept, grp):
  mesh = plsc.VectorSubcoreMesh(core_axis_name="c", subcore_axis_name="s",
                                num_cores=1)
  ngrp = cpt // grp

  def body(dst_hbm, deg_out, ones_v, didx_v, tmp_v, sem, accd):
    s = lax.axis_index("s")
    st = s * rpt
    one = jnp.full((LANES,), 1.0, jnp.float32)

    def fill_ones(i, c):
      ones_v[i, :] = one
      return c

    lax.fori_loop(0, grp * CH, fill_ones, 0)

    def fill_tmp(i, c):
      tmp_v[i, :] = one
      return c

    lax.fori_loop(0, rpt, fill_tmp, 0)
    pltpu.sync_copy(tmp_v, accd.at[pl.ds(st, rpt)])
    plsc.subcore_barrier()

    def group(j, c):
      base = s * ept + j * (grp * CH)
      pltpu.sync_copy(dst_hbm.at[pl.ds(base, grp * CH)], didx_v)
      pltpu.async_copy(ones_v, accd.at[didx_v], sem, add=True).wait()
      return c

    lax.fori_loop(0, ngrp, group, 0)
    plsc.subcore_barrier()
    pltpu.sync_copy(accd.at[pl.ds(st, rpt)], tmp_v)
    pltpu.sync_copy(tmp_v, deg_out.at[pl.ds(st, rpt)])

  return pl.kernel(
      body,
      out_type=jax.ShapeDtypeStruct((npad, LANES), jnp.float32),
      mesh=mesh,
      compiler_params=pltpu.CompilerParams(use_tc_tiling_on_sc=False),
      scratch_types=[
          pltpu.VMEM((grp * CH, LANES), jnp.float32),
          pltpu.VMEM((grp * CH,), jnp.int32),
          pltpu.VMEM((rpt, LANES), jnp.float32),
          pltpu.SemaphoreType.DMA,
          pltpu.VMEM_SHARED((npad, LANES), jnp.float32),
      ],
  )


def _mlp_call(npad, n, f_in, h, br):
  grid = npad // br

  def body(x_ref, w1t_ref, b1_ref, deg_ref, g0_ref, d2_ref):
    i = pl.program_id(0)
    h1 = jnp.dot(x_ref[...], w1t_ref[...],
                 preferred_element_type=jnp.float32) + b1_ref[...]
    h1 = jnp.maximum(h1, 0.0)
    deg = deg_ref[...]
    dinv = lax.rsqrt(deg[:, :1])
    rid = lax.broadcasted_iota(jnp.int32, (br, 1), 0) + i * br
    g0_ref[...] = jnp.where(rid < n, h1 * dinv, 0.0)
    d2_ref[...] = 1.0 / deg

  return pl.pallas_call(
      body,
      grid=(grid,),
      in_specs=[
          pl.BlockSpec((br, f_in), lambda i: (i, 0)),
          pl.BlockSpec((f_in, h), lambda i: (0, 0)),
          pl.BlockSpec((1, h), lambda i: (0, 0)),
          pl.BlockSpec((br, LANES), lambda i: (i, 0)),
      ],
      out_specs=[
          pl.BlockSpec((br, h), lambda i: (i, 0)),
          pl.BlockSpec((br, LANES), lambda i: (i, 0)),
      ],
      out_shape=[
          jax.ShapeDtypeStruct((npad, h), jnp.float32),
          jax.ShapeDtypeStruct((npad, LANES), jnp.float32),
      ],
  )


def _prop_call(npad, h, kk, rpt, cpt, ept, grp):
  mesh = plsc.VectorSubcoreMesh(core_axis_name="c", subcore_axis_name="s",
                                num_cores=1)
  hv = h // LANES
  gch = grp * CH
  ngrp = cpt // grp
  npair = ngrp // 2
  nsub = rpt // CH

  def body(g0_hbm, src_hbm, dst_hbm, d2_hbm, gflat, sidx_a, sidx_b, didx_a,
           didx_b, rows_a, rows_b, comb_v, d2_v, semg_a, semg_b, sems_a,
           sems_b, acc):
    s = lax.axis_index("s")
    st = s * rpt
    pltpu.sync_copy(d2_hbm.at[pl.ds(st, rpt)], d2_v)
    for rr in range(nsub):
      pltpu.sync_copy(g0_hbm.at[pl.ds(st + rr * CH, CH)], comb_v)
      pltpu.sync_copy(comb_v, gflat.at[pl.ds(st + rr * CH, CH)])
    plsc.subcore_barrier()

    dummy = gflat.at[pl.ds(0, gch)]

    for k in range(1, kk + 1):
      off_prev = (k - 1) * npad
      off_k = k * npad
      pltpu.sync_copy(gflat.at[pl.ds(off_prev + st, rpt)],
                      acc.at[pl.ds(st, rpt)])
      plsc.subcore_barrier()

      def fire_g(g, sidx, didx, rows, semg, off_prev=off_prev):
        base = s * ept + g * gch
        pltpu.sync_copy(src_hbm.at[pl.ds(base, gch)], sidx)
        pltpu.sync_copy(dst_hbm.at[pl.ds(base, gch)], didx)
        for t in range(gch // LANES):
          sl = pl.ds(t * LANES, LANES)
          sidx[sl] = sidx[sl] + off_prev
        pltpu.async_copy(gflat.at[sidx], rows, semg)

      def wait_dma(rows, sem):
        pltpu.make_async_copy(dummy, rows, sem).wait()

      def fire_s(didx, rows, sems):
        pltpu.async_copy(rows, acc.at[didx], sems, add=True)

      def pair(i, c):
        @pl.when(i > 0)
        def _():
          wait_dma(rows_a, sems_a)

        fire_g(2 * i, sidx_a, didx_a, rows_a, semg_a)

        @pl.when(i > 0)
        def _():
          wait_dma(rows_b, semg_b)
          fire_s(didx_b, rows_b, sems_b)
          wait_dma(rows_b, sems_b)

        fire_g(2 * i + 1, sidx_b, didx_b, rows_b, semg_b)
        wait_dma(rows_a, semg_a)
        fire_s(didx_a, rows_a, sems_a)
        return c

      lax.fori_loop(0, npair, pair, 0)
      wait_dma(rows_b, semg_b)
      fire_s(didx_b, rows_b, sems_b)
      wait_dma(rows_b, sems_b)
      wait_dma(rows_a, sems_a)
      plsc.subcore_barrier()

      for rr in range(nsub):
        pltpu.sync_copy(acc.at[pl.ds(st + rr * CH, CH)], comb_v)

        def comb(r, c, rr=rr):
          dv = d2_v[rr * CH + r, :]
          for t in range(hv):
            sl = pl.ds(t * LANES, LANES)
            comb_v[r, sl] = comb_v[r, sl] * dv
          return c

        lax.fori_loop(0, CH, comb, 0)
        pltpu.sync_copy(comb_v, gflat.at[pl.ds(off_k + st + rr * CH, CH)])
      plsc.subcore_barrier()

  return pl.kernel(
      body,
      out_type=jax.ShapeDtypeStruct(((kk + 1) * npad, h), jnp.float32),
      mesh=mesh,
      compiler_params=pltpu.CompilerParams(use_tc_tiling_on_sc=False),
      scratch_types=[
          pltpu.VMEM((gch,), jnp.int32),
          pltpu.VMEM((gch,), jnp.int32),
          pltpu.VMEM((gch,), jnp.int32),
          pltpu.VMEM((gch,), jnp.int32),
          pltpu.VMEM((gch, h), jnp.float32),
          pltpu.VMEM((gch, h), jnp.float32),
          pltpu.VMEM((CH, h), jnp.float32),
          pltpu.VMEM((rpt, LANES), jnp.float32),
          pltpu.SemaphoreType.DMA,
          pltpu.SemaphoreType.DMA,
          pltpu.SemaphoreType.DMA,
          pltpu.SemaphoreType.DMA,
          pltpu.VMEM_SHARED((npad, h), jnp.float32),
      ],
  )


def _out_call(npad, h, c, kk, br):
  grid = npad // br
  k1 = kk + 1

  def body(g_ref, deg_ref, temp_ref, w2t_ref, o_ref):
    acc = temp_ref[0] * g_ref[0]
    for k in range(1, k1):
      acc = acc + temp_ref[k] * g_ref[k]
    out = jnp.sqrt(deg_ref[...][:, :1]) * acc
    h2 = jnp.dot(out, w2t_ref[...], preferred_element_type=jnp.float32)
    m = jnp.max(h2, axis=1, keepdims=True)
    ex = jnp.exp(h2 - m)
    lse = jnp.log(jnp.sum(ex, axis=1, keepdims=True))
    o_ref[...] = (h2 - m) - lse

  return pl.pallas_call(
      body,
      grid=(grid,),
      in_specs=[
          pl.BlockSpec((k1, br, h), lambda i: (0, i, 0)),
          pl.BlockSpec((br, LANES), lambda i: (i, 0)),
          pl.BlockSpec(memory_space=pltpu.SMEM),
          pl.BlockSpec((h, c), lambda i: (0, 0)),
      ],
      out_specs=pl.BlockSpec((br, c), lambda i: (i, 0)),
      out_shape=jax.ShapeDtypeStruct((npad, c), jnp.float32),
  )


def kernel(x, edge_index, y_pred, W1, b1, W2, temp):
  n, f_in = x.shape
  h = W1.shape[0]
  c = W2.shape[0]
  kk = int(temp.shape[0]) - 1
  e = edge_index.shape[1]

  grp = 4
  rpt = _round_up(-(-n // NT), CH)
  npad = rpt * NT
  ept = _round_up(-(-e // NT), 2 * grp * CH)
  cpt = ept // CH
  epad = ept * NT

  pad_node = npad - 1
  epad_fill = jnp.full((epad - e,), pad_node, jnp.int32)
  perm = jnp.argsort(edge_index[0])
  src_p = jnp.concatenate([edge_index[0][perm].astype(jnp.int32), epad_fill])
  dst_p = jnp.concatenate([edge_index[1][perm].astype(jnp.int32), epad_fill])
  x_p = jnp.pad(x, ((0, npad - n), (0, 0)))
  w1t = W1.T
  w2t = W2.T
  b1r = b1.reshape(1, h)

  deg16 = _deg_call(npad, rpt, cpt, ept, grp)(dst_p)
  g0, d2 = _mlp_call(npad, n, f_in, h, rpt)(x_p, w1t, b1r, deg16)
  gflat = _prop_call(npad, h, kk, rpt, cpt, ept, grp)(g0, src_p, dst_p, d2)
  gk = gflat.reshape(kk + 1, npad, h)
  outp = _out_call(npad, h, c, kk, rpt)(gk, deg16, temp, w2t)
  return outp[:n]


# speedup vs baseline: 1.2039x; 1.2039x over previous
import functools

import jax
import jax.numpy as jnp
from jax import lax
from jax.experimental import pallas as pl
from jax.experimental.pallas import tpu as pltpu
from jax.experimental.pallas import tpu_sc as plsc

NT = 16
LANES = 16
CH = 128


def _round_up(a, b):
  return (a + b - 1) // b * b


def _deg_call(npad, rpt, cpt, ept, grp):
  mesh = plsc.VectorSubcoreMesh(core_axis_name="c", subcore_axis_name="s",
                                num_cores=1)
  ngrp = cpt // grp

  def body(dst_hbm, deg_out, ones_v, didx_v, tmp_v, sem, accd):
    s = lax.axis_index("s")
    st = s * rpt
    one = jnp.full((LANES,), 1.0, jnp.float32)

    def fill_ones(i, c):
      ones_v[i, :] = one
      return c

    lax.fori_loop(0, grp * CH, fill_ones, 0)

    def fill_tmp(i, c):
      tmp_v[i, :] = one
      return c

    lax.fori_loop(0, rpt, fill_tmp, 0)
    pltpu.sync_copy(tmp_v, accd.at[pl.ds(st, rpt)])
    plsc.subcore_barrier()

    def group(j, c):
      base = s * ept + j * (grp * CH)
      pltpu.sync_copy(dst_hbm.at[pl.ds(base, grp * CH)], didx_v)
      pltpu.async_copy(ones_v, accd.at[didx_v], sem, add=True).wait()
      return c

    lax.fori_loop(0, ngrp, group, 0)
    plsc.subcore_barrier()
    pltpu.sync_copy(accd.at[pl.ds(st, rpt)], tmp_v)
    pltpu.sync_copy(tmp_v, deg_out.at[pl.ds(st, rpt)])

  return pl.kernel(
      body,
      out_type=jax.ShapeDtypeStruct((npad, LANES), jnp.float32),
      mesh=mesh,
      compiler_params=pltpu.CompilerParams(use_tc_tiling_on_sc=False),
      scratch_types=[
          pltpu.VMEM((grp * CH, LANES), jnp.float32),
          pltpu.VMEM((grp * CH,), jnp.int32),
          pltpu.VMEM((rpt, LANES), jnp.float32),
          pltpu.SemaphoreType.DMA,
          pltpu.VMEM_SHARED((npad, LANES), jnp.float32),
      ],
  )


def _mlp_call(npad, n, f_in, h, br):
  grid = npad // br

  def body(x_ref, w1t_ref, b1_ref, deg_ref, g0_ref, d2_ref):
    i = pl.program_id(0)
    h1 = jnp.dot(x_ref[...], w1t_ref[...],
                 preferred_element_type=jnp.float32) + b1_ref[...]
    h1 = jnp.maximum(h1, 0.0)
    deg = deg_ref[...]
    dinv = lax.rsqrt(deg[:, :1])
    rid = lax.broadcasted_iota(jnp.int32, (br, 1), 0) + i * br
    g0_ref[...] = jnp.where(rid < n, h1 * dinv, 0.0)
    d2_ref[...] = 1.0 / deg

  return pl.pallas_call(
      body,
      grid=(grid,),
      in_specs=[
          pl.BlockSpec((br, f_in), lambda i: (i, 0)),
          pl.BlockSpec((f_in, h), lambda i: (0, 0)),
          pl.BlockSpec((1, h), lambda i: (0, 0)),
          pl.BlockSpec((br, LANES), lambda i: (i, 0)),
      ],
      out_specs=[
          pl.BlockSpec((br, h), lambda i: (i, 0)),
          pl.BlockSpec((br, LANES), lambda i: (i, 0)),
      ],
      out_shape=[
          jax.ShapeDtypeStruct((npad, h), jnp.float32),
          jax.ShapeDtypeStruct((npad, LANES), jnp.float32),
      ],
  )


def _prop_call(npad, h, kk, rpt, cpt, ept, grp):
  mesh = plsc.VectorSubcoreMesh(core_axis_name="c", subcore_axis_name="s",
                                num_cores=1)
  hv = h // LANES
  gch = grp * CH
  ngrp = cpt // grp
  npair = ngrp // 2
  nsub = rpt // CH

  def body(g0_hbm, src_hbm, dst_hbm, d2_hbm, gflat, sidx_a, sidx_b, didx_a,
           didx_b, rows_a, rows_b, comb_v, d2_v, semg_a, semg_b, sems_a,
           sems_b, acc):
    s = lax.axis_index("s")
    st = s * rpt
    pltpu.sync_copy(d2_hbm.at[pl.ds(st, rpt)], d2_v)
    for rr in range(nsub):
      pltpu.sync_copy(g0_hbm.at[pl.ds(st + rr * CH, CH)], comb_v)
      pltpu.sync_copy(comb_v, gflat.at[pl.ds(st + rr * CH, CH)])
    plsc.subcore_barrier()

    dummy = gflat.at[pl.ds(0, gch)]

    for k in range(1, kk + 1):
      off_prev = (k - 1) * npad
      off_k = k * npad
      pltpu.sync_copy(gflat.at[pl.ds(off_prev + st, rpt)],
                      acc.at[pl.ds(st, rpt)])
      plsc.subcore_barrier()

      def fire_g(g, sidx, didx, rows, semg, off_prev=off_prev):
        base = s * ept + g * gch
        pltpu.sync_copy(src_hbm.at[pl.ds(base, gch)], sidx)
        pltpu.sync_copy(dst_hbm.at[pl.ds(base, gch)], didx)
        for t in range(gch // LANES):
          sl = pl.ds(t * LANES, LANES)
          sidx[sl] = sidx[sl] + off_prev
        pltpu.async_copy(gflat.at[sidx], rows, semg)

      def wait_dma(rows, sem):
        pltpu.make_async_copy(dummy, rows, sem).wait()

      def fire_s(didx, rows, sems):
        pltpu.async_copy(rows, acc.at[didx], sems, add=True)

      def pair(i, c):
        @pl.when(i > 0)
        def _():
          wait_dma(rows_a, sems_a)

        fire_g(2 * i, sidx_a, didx_a, rows_a, semg_a)

        @pl.when(i > 0)
        def _():
          wait_dma(rows_b, semg_b)
          fire_s(didx_b, rows_b, sems_b)
          wait_dma(rows_b, sems_b)

        fire_g(2 * i + 1, sidx_b, didx_b, rows_b, semg_b)
        wait_dma(rows_a, semg_a)
        fire_s(didx_a, rows_a, sems_a)
        return c

      lax.fori_loop(0, npair, pair, 0)
      wait_dma(rows_b, semg_b)
      fire_s(didx_b, rows_b, sems_b)
      wait_dma(rows_b, sems_b)
      wait_dma(rows_a, sems_a)
      plsc.subcore_barrier()

      for rr in range(nsub):
        pltpu.sync_copy(acc.at[pl.ds(st + rr * CH, CH)], comb_v)

        def comb(r, c, rr=rr):
          dv = d2_v[rr * CH + r, :]
          for t in range(hv):
            sl = pl.ds(t * LANES, LANES)
            comb_v[r, sl] = comb_v[r, sl] * dv
          return c

        lax.fori_loop(0, CH, comb, 0)
        pltpu.sync_copy(comb_v, gflat.at[pl.ds(off_k + st + rr * CH, CH)])
      plsc.subcore_barrier()

  return pl.kernel(
      body,
      out_type=jax.ShapeDtypeStruct(((kk + 1) * npad, h), jnp.float32),
      mesh=mesh,
      compiler_params=pltpu.CompilerParams(use_tc_tiling_on_sc=False),
      scratch_types=[
          pltpu.VMEM((gch,), jnp.int32),
          pltpu.VMEM((gch,), jnp.int32),
          pltpu.VMEM((gch,), jnp.int32),
          pltpu.VMEM((gch,), jnp.int32),
          pltpu.VMEM((gch, h), jnp.float32),
          pltpu.VMEM((gch, h), jnp.float32),
          pltpu.VMEM((CH, h), jnp.float32),
          pltpu.VMEM((rpt, LANES), jnp.float32),
          pltpu.SemaphoreType.DMA,
          pltpu.SemaphoreType.DMA,
          pltpu.SemaphoreType.DMA,
          pltpu.SemaphoreType.DMA,
          pltpu.VMEM_SHARED((npad, h), jnp.float32),
      ],
  )


def _out_call(npad, h, c, kk, br):
  grid = npad // br
  k1 = kk + 1

  def body(g_ref, deg_ref, temp_ref, w2t_ref, o_ref):
    acc = temp_ref[0] * g_ref[0]
    for k in range(1, k1):
      acc = acc + temp_ref[k] * g_ref[k]
    out = jnp.sqrt(deg_ref[...][:, :1]) * acc
    h2 = jnp.dot(out, w2t_ref[...], preferred_element_type=jnp.float32)
    m = jnp.max(h2, axis=1, keepdims=True)
    ex = jnp.exp(h2 - m)
    lse = jnp.log(jnp.sum(ex, axis=1, keepdims=True))
    o_ref[...] = (h2 - m) - lse

  return pl.pallas_call(
      body,
      grid=(grid,),
      in_specs=[
          pl.BlockSpec((k1, br, h), lambda i: (0, i, 0)),
          pl.BlockSpec((br, LANES), lambda i: (i, 0)),
          pl.BlockSpec(memory_space=pltpu.SMEM),
          pl.BlockSpec((h, c), lambda i: (0, 0)),
      ],
      out_specs=pl.BlockSpec((br, c), lambda i: (i, 0)),
      out_shape=jax.ShapeDtypeStruct((npad, c), jnp.float32),
  )


def kernel(x, edge_index, y_pred, W1, b1, W2, temp):
  n, f_in = x.shape
  h = W1.shape[0]
  c = W2.shape[0]
  kk = int(temp.shape[0]) - 1
  e = edge_index.shape[1]

  grp = 4
  rpt = _round_up(-(-n // NT), CH)
  npad = rpt * NT
  ept = _round_up(-(-e // NT), 2 * grp * CH)
  cpt = ept // CH
  epad = ept * NT

  pad_node = npad - 1
  epad_fill = jnp.full((epad - e,), pad_node, jnp.int32)
  src_p = jnp.concatenate([edge_index[0].astype(jnp.int32), epad_fill])
  dst_p = jnp.concatenate([edge_index[1].astype(jnp.int32), epad_fill])
  x_p = jnp.pad(x, ((0, npad - n), (0, 0)))
  w1t = W1.T
  w2t = W2.T
  b1r = b1.reshape(1, h)

  deg16 = _deg_call(npad, rpt, cpt, ept, grp)(dst_p)
  g0, d2 = _mlp_call(npad, n, f_in, h, rpt)(x_p, w1t, b1r, deg16)
  gflat = _prop_call(npad, h, kk, rpt, cpt, ept, grp)(g0, src_p, dst_p, d2)
  gk = gflat.reshape(kk + 1, npad, h)
  outp = _out_call(npad, h, c, kk, rpt)(gk, deg16, temp, w2t)
  return outp[:n]
